# baseline (speedup 1.0000x reference)
import functools

import jax
import jax.numpy as jnp
from jax import lax
from jax.experimental import pallas as pl
from jax.experimental.pallas import tpu as pltpu
from jax.experimental.pallas import tpu_sc as plsc

_BATCH = 16384
_NCLS = 1000
_LANES = 16
_NCORES = 2
_NSUB = 16
_NTILES = _NCORES * _NSUB
_COLS_PER_TILE = _BATCH // _NTILES
_NGRP = _COLS_PER_TILE // _LANES
_CCHUNK = 120
_CLAST = _NCLS - 8 * _CCHUNK
_UNROLL = 8


def _onehot_body(labels_hbm, out_hbm, labels_v, buf0, buf1, sem0, sem1, seml):
    wid = lax.axis_index("s") * _NCORES + lax.axis_index("c")
    base = wid * _COLS_PER_TILE

    lbl_copy = pltpu.async_copy(
        labels_hbm.at[pl.ds(base, _COLS_PER_TILE)], labels_v, seml
    )

    zeros16 = jnp.zeros((_LANES,), jnp.float32)
    ones16 = jnp.ones((_LANES,), jnp.float32)
    iota16 = lax.iota(jnp.int32, _LANES)

    def _zero_buf(buf):
        def _row(r, carry):
            for off in range(0, _COLS_PER_TILE, _LANES):
                buf[r, pl.ds(off, _LANES)] = zeros16
            return carry

        lax.fori_loop(0, _CCHUNK, _row, 0)

    def _scatter_pass(buf, c0_set, c0_restore, set_size=_CCHUNK):
        def _step(i, carry):
            for j in range(_UNROLL):
                g = i * _UNROLL + j
                col = g * _LANES + iota16
                lbl = labels_v[pl.ds(g * _LANES, _LANES)]
                if c0_restore is not None:
                    rel = lbl - c0_restore
                    m = rel.astype(jnp.uint32) < jnp.uint32(_CCHUNK)
                    plsc.store_scatter(buf, [rel, col], zeros16, mask=m)
                rel = lbl - c0_set
                m = rel.astype(jnp.uint32) < jnp.uint32(set_size)
                plsc.store_scatter(buf, [rel, col], ones16, mask=m)
            return carry

        lax.fori_loop(0, _NGRP // _UNROLL, _step, 0)

    def _dma(buf, c0, sem, rows=_CCHUNK):
        return pltpu.async_copy(
            buf.at[pl.ds(0, rows)] if rows != _CCHUNK else buf,
            out_hbm.at[pl.ds(c0, rows), pl.ds(base, _COLS_PER_TILE)],
            sem,
        )

    def _wait(buf, sem):
        pltpu.make_async_copy(
            buf,
            out_hbm.at[pl.ds(0, _CCHUNK), pl.ds(base, _COLS_PER_TILE)],
            sem,
        ).wait()

    _zero_buf(buf0)
    lbl_copy.wait()
    _scatter_pass(buf0, 0, None)
    _dma(buf0, 0, sem0)
    _zero_buf(buf1)
    _scatter_pass(buf1, _CCHUNK, None)
    _dma(buf1, _CCHUNK, sem1)

    def _pair(p, carry):
        c0 = 2 * p * _CCHUNK
        _wait(buf0, sem0)
        _scatter_pass(buf0, c0, c0 - 2 * _CCHUNK)
        _dma(buf0, c0, sem0)
        _wait(buf1, sem1)
        _scatter_pass(buf1, c0 + _CCHUNK, c0 - _CCHUNK)
        _dma(buf1, c0 + _CCHUNK, sem1)
        return carry

    lax.fori_loop(1, 4, _pair, 0)

    c_last = 8 * _CCHUNK
    _wait(buf0, sem0)
    _scatter_pass(buf0, c_last, c_last - 2 * _CCHUNK, set_size=_CLAST)
    pltpu.async_copy(
        buf0.at[pl.ds(0, _CLAST)],
        out_hbm.at[pl.ds(c_last, _CLAST), pl.ds(base, _COLS_PER_TILE)],
        sem0,
    ).wait()
    _wait(buf1, sem1)


_onehot_t = functools.partial(
    pl.kernel,
    out_type=jax.ShapeDtypeStruct((_NCLS, _BATCH), jnp.float32),
    mesh=plsc.VectorSubcoreMesh(core_axis_name="c", subcore_axis_name="s"),
    compiler_params=pltpu.CompilerParams(
        needs_layout_passes=False, use_tc_tiling_on_sc=True
    ),
    scratch_types=[
        pltpu.VMEM((_COLS_PER_TILE,), jnp.int32),
        pltpu.VMEM((_CCHUNK, _COLS_PER_TILE), jnp.float32),
        pltpu.VMEM((_CCHUNK, _COLS_PER_TILE), jnp.float32),
        pltpu.SemaphoreType.DMA,
        pltpu.SemaphoreType.DMA,
        pltpu.SemaphoreType.DMA,
    ],
)(_onehot_body)


def kernel(labels):
    return _onehot_t(labels.astype(jnp.int32)).T


# speedup vs baseline: 1.0395x; 1.0395x over previous
import functools

import jax
import jax.numpy as jnp
from jax import lax
from jax.experimental import pallas as pl
from jax.experimental.pallas import tpu as pltpu
from jax.experimental.pallas import tpu_sc as plsc

_BATCH = 16384
_NCLS = 1000
_LANES = 16
_NCORES = 2
_NSUB = 16
_NTILES = _NCORES * _NSUB
_COLS_PER_TILE = _BATCH // _NTILES
_NGRP = _COLS_PER_TILE // _LANES
_CCHUNK = 40
_NCHUNK = _NCLS // _CCHUNK
_UNROLL = 8


def _onehot_body(labels_hbm, out_hbm, labels_v, buf0, buf1, sem0, sem1, seml):
    wid = lax.axis_index("s") * _NCORES + lax.axis_index("c")
    base = wid * _COLS_PER_TILE

    lbl_copy = pltpu.async_copy(
        labels_hbm.at[pl.ds(base, _COLS_PER_TILE)], labels_v, seml
    )

    zeros16 = jnp.zeros((_LANES,), jnp.float32)
    ones16 = jnp.ones((_LANES,), jnp.float32)
    iota16 = lax.iota(jnp.int32, _LANES)

    def _zero_buf(buf):
        def _row(r, carry):
            for off in range(0, _COLS_PER_TILE, _LANES):
                buf[r, pl.ds(off, _LANES)] = zeros16
            return carry

        lax.fori_loop(0, _CCHUNK, _row, 0)

    def _scatter_pass(buf, c0_set, c0_restore):
        def _step(i, carry):
            for j in range(_UNROLL):
                g = i * _UNROLL + j
                col = g * _LANES + iota16
                lbl = labels_v[pl.ds(g * _LANES, _LANES)]
                if c0_restore is not None:
                    rel = lbl - c0_restore
                    m = rel.astype(jnp.uint32) < jnp.uint32(_CCHUNK)
                    plsc.store_scatter(buf, [rel, col], zeros16, mask=m)
                rel = lbl - c0_set
                m = rel.astype(jnp.uint32) < jnp.uint32(_CCHUNK)
                plsc.store_scatter(buf, [rel, col], ones16, mask=m)
            return carry

        lax.fori_loop(0, _NGRP // _UNROLL, _step, 0)

    def _dma(buf, c0, sem):
        return pltpu.async_copy(
            buf,
            out_hbm.at[pl.ds(c0, _CCHUNK), pl.ds(base, _COLS_PER_TILE)],
            sem,
        )

    def _wait(buf, sem):
        pltpu.make_async_copy(
            buf,
            out_hbm.at[pl.ds(0, _CCHUNK), pl.ds(base, _COLS_PER_TILE)],
            sem,
        ).wait()

    _zero_buf(buf0)
    lbl_copy.wait()
    _scatter_pass(buf0, 0, None)
    _dma(buf0, 0, sem0)
    _zero_buf(buf1)
    _scatter_pass(buf1, _CCHUNK, None)
    _dma(buf1, _CCHUNK, sem1)

    def _pair(p, carry):
        c0 = 2 * p * _CCHUNK
        _wait(buf0, sem0)
        _scatter_pass(buf0, c0, c0 - 2 * _CCHUNK)
        _dma(buf0, c0, sem0)
        _wait(buf1, sem1)
        _scatter_pass(buf1, c0 + _CCHUNK, c0 - _CCHUNK)
        _dma(buf1, c0 + _CCHUNK, sem1)
        return carry

    lax.fori_loop(1, (_NCHUNK - 1) // 2, _pair, 0)

    c_last = (_NCHUNK - 1) * _CCHUNK
    _wait(buf0, sem0)
    _scatter_pass(buf0, c_last, c_last - 2 * _CCHUNK)
    _dma(buf0, c_last, sem0).wait()
    _wait(buf1, sem1)


_onehot_t = functools.partial(
    pl.kernel,
    out_type=jax.ShapeDtypeStruct((_NCLS, _BATCH), jnp.float32),
    mesh=plsc.VectorSubcoreMesh(core_axis_name="c", subcore_axis_name="s"),
    compiler_params=pltpu.CompilerParams(
        needs_layout_passes=False, use_tc_tiling_on_sc=True
    ),
    scratch_types=[
        pltpu.VMEM((_COLS_PER_TILE,), jnp.int32),
        pltpu.VMEM((_CCHUNK, _COLS_PER_TILE), jnp.float32),
        pltpu.VMEM((_CCHUNK, _COLS_PER_TILE), jnp.float32),
        pltpu.SemaphoreType.DMA,
        pltpu.SemaphoreType.DMA,
        pltpu.SemaphoreType.DMA,
    ],
)(_onehot_body)


def kernel(labels):
    return _onehot_t(labels.astype(jnp.int32)).T
